# baseline (speedup 1.0000x reference)
import jax
import jax.numpy as jnp
from jax import lax
from jax.experimental import pallas as pl
from jax.experimental.pallas import tpu as pltpu
from jax.experimental.pallas import tpu_sc as plsc

K = 8192
D = 49
N = 4
NS = 16
CHUNK = K // NS
GROUPS = CHUNK // 16


def _sc_body(x_hbm, cbf_hbm, cb2_hbm, out_hbm,
             cb_v, x_v, stage_v, best_sh, all_v, rows_v):
    c = lax.axis_index("c")
    s = lax.axis_index("s")

    pltpu.sync_copy(cbf_hbm.at[pl.ds(s * (CHUNK * D), CHUNK * D)], cb_v)
    pltpu.sync_copy(x_hbm, x_v)

    q0 = 2 * c
    q1 = 2 * c + 1
    lane = lax.iota(jnp.int32, 16)
    lane_d = lane * D
    big = jnp.float32(jnp.inf)

    def xchunks(q):
        return [x_v[q, pl.ds(0, 16)], x_v[q, pl.ds(16, 16)],
                x_v[q, pl.ds(32, 16)], x_v[q, pl.ds(33, 16)]]

    xq0 = xchunks(q0)
    xq1 = xchunks(q1)

    def xscal(xq, d):
        if d < 48:
            return xq[d // 16][d % 16]
        return xq[3][15]

    def group_body(g, carry):
        bv0, bi0, bv1, bi1 = carry
        base = lane_d + g * (16 * D)
        cn = jnp.zeros((16,), jnp.float32)
        d0 = jnp.zeros((16,), jnp.float32)
        d1 = jnp.zeros((16,), jnp.float32)
        for d in range(D):
            cvec = plsc.load_gather(cb_v, [base + d])
            cn = cn + cvec * cvec
            d0 = d0 + cvec * xscal(xq0, d)
            d1 = d1 + cvec * xscal(xq1, d)
        dist0 = cn - 2.0 * d0
        dist1 = cn - 2.0 * d1
        rows = s * CHUNK + g * 16 + lane
        upd0 = dist0 < bv0
        upd1 = dist1 < bv1
        bv0 = jnp.where(upd0, dist0, bv0)
        bi0 = jnp.where(upd0, rows, bi0)
        bv1 = jnp.where(upd1, dist1, bv1)
        bi1 = jnp.where(upd1, rows, bi1)
        return bv0, bi0, bv1, bi1

    init = (jnp.full((16,), big), jnp.zeros((16,), jnp.int32),
            jnp.full((16,), big), jnp.zeros((16,), jnp.int32))
    bv0, bi0, bv1, bi1 = lax.fori_loop(0, GROUPS, group_body, init)

    stage_v[pl.ds(0, 16)] = bv0
    stage_v[pl.ds(16, 16)] = bv1
    stage_v[pl.ds(32, 16)] = plsc.bitcast(bi0, jnp.float32)
    stage_v[pl.ds(48, 16)] = plsc.bitcast(bi1, jnp.float32)
    pltpu.sync_copy(stage_v, best_sh.at[s])
    plsc.subcore_barrier()

    @pl.when(s == 0)
    def _finalize():
        pltpu.sync_copy(best_sh, all_v)
        winners = []
        for q in range(2):
            bv = all_v[0, pl.ds(q * 16, 16)]
            bi = plsc.bitcast(all_v[0, pl.ds(32 + q * 16, 16)], jnp.int32)
            for i in range(1, NS):
                v = all_v[i, pl.ds(q * 16, 16)]
                ix = plsc.bitcast(all_v[i, pl.ds(32 + q * 16, 16)], jnp.int32)
                upd = v < bv
                bv = jnp.where(upd, v, bv)
                bi = jnp.where(upd, ix, bi)
            m = jnp.min(bv)
            cand = jnp.where(bv == m, bi, jnp.int32(K))
            winners.append(jnp.min(cand))
        pltpu.sync_copy(cb2_hbm.at[pl.ds(winners[0], 1)], rows_v.at[pl.ds(0, 1)])
        pltpu.sync_copy(cb2_hbm.at[pl.ds(winners[1], 1)], rows_v.at[pl.ds(1, 1)])
        pltpu.sync_copy(rows_v.at[pl.ds(0, 2)], out_hbm.at[pl.ds(2 * c, 2)])


def kernel(x, codebook):
    cbf = jnp.reshape(codebook, (K * D,))
    mesh = plsc.VectorSubcoreMesh(core_axis_name="c", subcore_axis_name="s",
                                  num_cores=2, num_subcores=NS)
    call = pl.kernel(
        _sc_body,
        out_type=jax.ShapeDtypeStruct((N, D), jnp.float32),
        mesh=mesh,
        compiler_params=pltpu.CompilerParams(needs_layout_passes=False),
        scratch_types=[
            pltpu.VMEM((CHUNK * D,), jnp.float32),
            pltpu.VMEM((N, D), jnp.float32),
            pltpu.VMEM((128,), jnp.float32),
            pltpu.VMEM_SHARED((NS, 128), jnp.float32),
            pltpu.VMEM((NS, 128), jnp.float32),
            pltpu.VMEM((2, D), jnp.float32),
        ],
    )
    out = call(x, cbf, codebook)
    return jnp.reshape(out, (4, 7, 7))


# speedup vs baseline: 3.6163x; 3.6163x over previous
import jax
import jax.numpy as jnp
from jax.experimental import pallas as pl
from jax.experimental.pallas import tpu as pltpu

K = 8192
D = 49
N = 4


def _vq_body(x_ref, cb_ref, out_ref):
    xs = x_ref[...]
    cb = cb_ref[...]
    b2r = jax.lax.dot_general(
        jnp.ones((1, D), jnp.float32), cb * cb, (((1,), (1,)), ((), ())),
        preferred_element_type=jnp.float32)
    dots = jax.lax.dot_general(
        xs, cb, (((1,), (1,)), ((), ())),
        preferred_element_type=jnp.float32)
    dist = b2r - 2.0 * dots
    idx = jnp.argmin(dist, axis=1)
    onehot = (jax.lax.broadcasted_iota(jnp.int32, (N, K), 1)
              == idx[:, None]).astype(jnp.float32)
    zq = jax.lax.dot_general(
        onehot, cb, (((1,), (0,)), ((), ())),
        preferred_element_type=jnp.float32)
    out_ref[...] = xs + (zq - xs)


def kernel(x, codebook):
    out = pl.pallas_call(
        _vq_body,
        out_shape=jax.ShapeDtypeStruct((N, D), jnp.float32),
    )(x, codebook)
    return jnp.reshape(out, (4, 7, 7))
